# Initial kernel scaffold; baseline (speedup 1.0000x reference)
import jax
import jax.numpy as jnp
from jax.experimental import pallas as pl


def kernel(features, rois):
    raise NotImplementedError("write your pallas kernel here")


# speedup vs baseline: 9.7007x; 9.7007x over previous
import functools

import jax
import jax.numpy as jnp
from jax import lax
from jax.experimental import pallas as pl
from jax.experimental.pallas import tpu as pltpu
from jax.experimental.pallas import tpu_sc as plsc

OUT_SIZE = 7
SPATIAL_SCALE = 0.25
SAMPLE_NUM = 2
L = 16


_GATHER_DNUMS = lax.GatherDimensionNumbers(
    offset_dims=(), collapsed_slice_dims=(0,), start_index_map=(0,))


def _dyn_gather(v, idx):
    return lax.gather(v, idx[:, None], _GATHER_DNUMS, (1,),
                      mode=lax.GatherScatterMode.PROMISE_IN_BOUNDS)


def _splat(v, i):
    return _dyn_gather(v, jnp.zeros((L,), jnp.int32) + i)


def _transpose_kernel(f_ref, o_ref):
    o_ref[...] = f_ref[...].T


def _make_table(features):
    N, C, H, W = features.shape
    f2 = features.reshape(N, C, H * W)
    out = pl.pallas_call(
        _transpose_kernel,
        grid=(N,),
        in_specs=[pl.BlockSpec((None, C, H * W), lambda b: (b, 0, 0))],
        out_specs=pl.BlockSpec((None, H * W, C), lambda b: (b, 0, 0)),
        out_shape=jax.ShapeDtypeStruct((N, H * W, C), jnp.float32),
    )(f2)
    return out.reshape(N * H * W, C)


def _roi_align_sc(table, rois_flat, *, N, C, H, W, KPAD):
    PH = PW = OUT_SIZE
    G = SAMPLE_NUM
    SX = PW * G
    SY = PH * G
    NS = SY * SX
    NTAP = 4 * NS
    CHUNK = 112
    NCHUNK = NTAP // CHUNK
    OUTSZ = C * PH * PW

    info = plsc.get_sparse_core_info()
    NC, NSUB = info.num_cores, info.num_subcores
    NW = NC * NSUB
    RPW = KPAD // NW
    mesh = plsc.VectorSubcoreMesh(core_axis_name="c", subcore_axis_name="s")

    @functools.partial(
        pl.kernel,
        mesh=mesh,
        compiler_params=pltpu.CompilerParams(
            needs_layout_passes=False, use_tc_tiling_on_sc=False),
        out_type=jax.ShapeDtypeStruct((KPAD * OUTSZ,), jnp.float32),
        scratch_types=[
            pltpu.VMEM((RPW * 8 + L,), jnp.float32),
            pltpu.VMEM((NTAP,), jnp.int32),
            pltpu.VMEM((NTAP, C), jnp.float32),
            pltpu.VMEM((OUTSZ,), jnp.float32),
            pltpu.SemaphoreType.DMA,
        ],
    )
    def k(table_hbm, rois_hbm, out_hbm, roipar, idxbuf, rows, outbuf, sem):
        wid = lax.axis_index("s") * NC + lax.axis_index("c")
        li = lax.iota(jnp.int32, L)
        lf = li.astype(jnp.float32)
        mask14 = li < SX
        zeros16 = jnp.zeros((L,), jnp.float32)
        off = (li >> 1).astype(jnp.float32) + (li & 1).astype(jnp.float32) * 0.5 + 0.25
        cols = [li + g * L for g in range(4)]
        cgs = [li * (PH * PW) + g * L * (PH * PW) for g in range(4)]

        pltpu.sync_copy(rois_hbm.at[pl.ds(wid * RPW * 8, RPW * 8)],
                        roipar.at[pl.ds(0, RPW * 8)])

        def per_roi(r, carry):
            pv = roipar[pl.ds(r * 8, L)]
            b = _splat(pv, 0)
            x1s = _splat(pv, 1) * SPATIAL_SCALE
            y1s = _splat(pv, 2) * SPATIAL_SCALE
            x2s = _splat(pv, 3) * SPATIAL_SCALE
            y2s = _splat(pv, 4) * SPATIAL_SCALE
            roi_w = jnp.maximum(x2s - x1s, 1.0)
            roi_h = jnp.maximum(y2s - y1s, 1.0)
            bin_w = roi_w / PW
            bin_h = roi_h / PH
            Y = y1s + off * bin_h
            X = x1s + off * bin_w
            vy = jnp.where((Y >= -1.0) & (Y <= float(H)), 0.5, 0.0)
            vx = jnp.where((X >= -1.0) & (X <= float(W)), 0.5, 0.0)
            yc = jnp.clip(Y, 0.0, float(H - 1))
            xc = jnp.clip(X, 0.0, float(W - 1))
            yi = jnp.minimum(yc.astype(jnp.int32), H - 2)
            xi = jnp.minimum(xc.astype(jnp.int32), W - 2)
            fy = yc - yi.astype(jnp.float32)
            fx = xc - xi.astype(jnp.float32)
            ay0 = (1.0 - fy) * vy
            ay1 = fy * vy
            bx0 = (1.0 - fx) * vx
            bx1 = fx * vx
            ybase_f = b * float(H * W) + yi.astype(jnp.float32) * float(W)

            def build(sy, c2):
                ybs = _splat(ybase_f, sy).astype(jnp.int32)
                i00 = ybs + xi
                pos = li + sy * SX
                plsc.store_scatter(idxbuf, [pos], i00, mask=mask14)
                plsc.store_scatter(idxbuf, [pos + NS], i00 + 1, mask=mask14)
                plsc.store_scatter(idxbuf, [pos + 2 * NS], i00 + W, mask=mask14)
                plsc.store_scatter(idxbuf, [pos + 3 * NS], i00 + W + 1, mask=mask14)
                return c2

            lax.fori_loop(0, SY, build, 0, unroll=False)

            copies = [
                pltpu.async_copy(
                    table_hbm.at[idxbuf.at[pl.ds(j * CHUNK, CHUNK)]],
                    rows.at[pl.ds(j * CHUNK, CHUNK)],
                    sem,
                )
                for j in range(NCHUNK)
            ]

            def zero(i, c2):
                for kk in range(SX):
                    outbuf[pl.ds(i * (SX * L) + kk * L, L)] = zeros16
                return c2

            lax.fori_loop(0, OUTSZ // (SX * L), zero, 0, unroll=False)

            for cp in copies:
                cp.wait()

            def accum(sy, c2):
                syv = jnp.zeros((L,), jnp.int32) + sy
                ay0s = _dyn_gather(ay0, syv)
                ay1s = _dyn_gather(ay1, syv)
                bin_base = (sy >> 1) * PW
                row0 = sy * SX
                for sx in range(SX):
                    bx0s = _splat(bx0, sx)
                    bx1s = _splat(bx1, sx)
                    w00 = ay0s * bx0s
                    w01 = ay0s * bx1s
                    w10 = ay1s * bx0s
                    w11 = ay1s * bx1s
                    bin_ = bin_base + (sx >> 1)
                    s0 = jnp.zeros((L,), jnp.int32) + (row0 + sx)
                    srows = [s0 + t * NS for t in range(4)]
                    for g in range(4):
                        v00 = plsc.load_gather(rows, [srows[0], cols[g]])
                        v01 = plsc.load_gather(rows, [srows[1], cols[g]])
                        v10 = plsc.load_gather(rows, [srows[2], cols[g]])
                        v11 = plsc.load_gather(rows, [srows[3], cols[g]])
                        acc = w00 * v00 + w01 * v01 + w10 * v10 + w11 * v11
                        plsc.addupdate_scatter(outbuf, [cgs[g] + bin_], acc)
                return c2

            lax.fori_loop(0, SY, accum, 0, unroll=False)

            roi = wid * RPW + r
            pltpu.sync_copy(outbuf, out_hbm.at[pl.ds(roi * OUTSZ, OUTSZ)])
            return carry

        lax.fori_loop(0, RPW, per_roi, 0, unroll=False)

    return k(table, rois_flat)


def kernel(features, rois):
    N, C, H, W = features.shape
    K = rois.shape[0]
    KPAD = 1024 if K <= 1024 else ((K + 255) // 256) * 256
    table = _make_table(features)
    rois_pad = jnp.zeros((KPAD, 8), jnp.float32).at[:K, :5].set(rois)
    out_flat = _roi_align_sc(table, rois_pad.reshape(-1),
                             N=N, C=C, H=H, W=W, KPAD=KPAD)
    out = out_flat.reshape(KPAD, C, OUT_SIZE, OUT_SIZE)
    return out[:K]


# speedup vs baseline: 9.7317x; 1.0032x over previous
import functools

import jax
import jax.numpy as jnp
from jax import lax
from jax.experimental import pallas as pl
from jax.experimental.pallas import tpu as pltpu
from jax.experimental.pallas import tpu_sc as plsc

OUT_SIZE = 7
SPATIAL_SCALE = 0.25
SAMPLE_NUM = 2
L = 16


_GATHER_DNUMS = lax.GatherDimensionNumbers(
    offset_dims=(), collapsed_slice_dims=(0,), start_index_map=(0,))


def _dyn_gather(v, idx):
    return lax.gather(v, idx[:, None], _GATHER_DNUMS, (1,),
                      mode=lax.GatherScatterMode.PROMISE_IN_BOUNDS)


def _splat(v, i):
    return _dyn_gather(v, jnp.zeros((L,), jnp.int32) + i)


def _transpose_kernel(f_ref, o_ref, *, HW, W):
    t = f_ref[...].T
    o_ref[:, : t.shape[1]] = t
    o_ref[: HW - W, t.shape[1]:] = t[W:, :]
    o_ref[HW - W:, t.shape[1]:] = t[HW - W:, :]


def _make_table(features):
    N, C, H, W = features.shape
    f2 = features.reshape(N, C, H * W)
    out = pl.pallas_call(
        functools.partial(_transpose_kernel, HW=H * W, W=W),
        grid=(N,),
        in_specs=[pl.BlockSpec((None, C, H * W), lambda b: (b, 0, 0))],
        out_specs=pl.BlockSpec((None, H * W, 2 * C), lambda b: (b, 0, 0)),
        out_shape=jax.ShapeDtypeStruct((N, H * W, 2 * C), jnp.float32),
    )(f2)
    return out.reshape(N * H * W, 2 * C)


def _roi_align_sc(table, rois_flat, *, N, C, H, W, KPAD):
    PH = PW = OUT_SIZE
    G = SAMPLE_NUM
    SX = PW * G
    SY = PH * G
    NS = SY * SX
    NTAP = 2 * NS
    CHUNKS = [(0, 112), (112, 112), (224, 112), (336, 56)]
    OUTSZ = C * PH * PW

    info = plsc.get_sparse_core_info()
    NC, NSUB = info.num_cores, info.num_subcores
    NW = NC * NSUB
    RPW = KPAD // NW
    mesh = plsc.VectorSubcoreMesh(core_axis_name="c", subcore_axis_name="s")

    @functools.partial(
        pl.kernel,
        mesh=mesh,
        compiler_params=pltpu.CompilerParams(
            needs_layout_passes=False, use_tc_tiling_on_sc=True),
        out_type=jax.ShapeDtypeStruct((KPAD * OUTSZ,), jnp.float32),
        scratch_types=[
            pltpu.VMEM((RPW * 8 + L,), jnp.float32),
            pltpu.VMEM((NTAP,), jnp.int32),
            pltpu.VMEM((NTAP, 2 * C), jnp.float32),
            pltpu.VMEM((OUTSZ,), jnp.float32),
            pltpu.SemaphoreType.DMA,
        ],
    )
    def k(table_hbm, rois_hbm, out_hbm, roipar, idxbuf, rows, outbuf, sem):
        wid = lax.axis_index("s") * NC + lax.axis_index("c")
        li = lax.iota(jnp.int32, L)
        lf = li.astype(jnp.float32)
        mask14 = li < SX
        zeros16 = jnp.zeros((L,), jnp.float32)
        off = (li >> 1).astype(jnp.float32) + (li & 1).astype(jnp.float32) * 0.5 + 0.25
        cols = [li + g * L for g in range(4)]
        cgs = [li * (PH * PW) + g * L * (PH * PW) for g in range(4)]

        pltpu.sync_copy(rois_hbm.at[pl.ds(wid * RPW * 8, RPW * 8)],
                        roipar.at[pl.ds(0, RPW * 8)])

        def per_roi(r, carry):
            pv = roipar[pl.ds(r * 8, L)]
            b = _splat(pv, 0)
            x1s = _splat(pv, 1) * SPATIAL_SCALE
            y1s = _splat(pv, 2) * SPATIAL_SCALE
            x2s = _splat(pv, 3) * SPATIAL_SCALE
            y2s = _splat(pv, 4) * SPATIAL_SCALE
            roi_w = jnp.maximum(x2s - x1s, 1.0)
            roi_h = jnp.maximum(y2s - y1s, 1.0)
            bin_w = roi_w / PW
            bin_h = roi_h / PH
            Y = y1s + off * bin_h
            X = x1s + off * bin_w
            vy = jnp.where((Y >= -1.0) & (Y <= float(H)), 0.5, 0.0)
            vx = jnp.where((X >= -1.0) & (X <= float(W)), 0.5, 0.0)
            yc = jnp.clip(Y, 0.0, float(H - 1))
            xc = jnp.clip(X, 0.0, float(W - 1))
            yi = jnp.minimum(yc.astype(jnp.int32), H - 2)
            xi = jnp.minimum(xc.astype(jnp.int32), W - 2)
            fy = yc - yi.astype(jnp.float32)
            fx = xc - xi.astype(jnp.float32)
            ay0 = (1.0 - fy) * vy
            ay1 = fy * vy
            bx0 = (1.0 - fx) * vx
            bx1 = fx * vx
            ybase_f = b * float(H * W) + yi.astype(jnp.float32) * float(W)

            def build(sy, c2):
                ybs = _splat(ybase_f, sy).astype(jnp.int32)
                i00 = ybs + xi
                pos = li + sy * SX
                plsc.store_scatter(idxbuf, [pos], i00, mask=mask14)
                plsc.store_scatter(idxbuf, [pos + NS], i00 + 1, mask=mask14)
                return c2

            lax.fori_loop(0, SY, build, 0, unroll=False)

            copies = [
                pltpu.async_copy(
                    table_hbm.at[idxbuf.at[pl.ds(off, sz)]],
                    rows.at[pl.ds(off, sz)],
                    sem,
                )
                for off, sz in CHUNKS
            ]

            def zero(i, c2):
                for kk in range(SX):
                    outbuf[pl.ds(i * (SX * L) + kk * L, L)] = zeros16
                return c2

            lax.fori_loop(0, OUTSZ // (SX * L), zero, 0, unroll=False)

            for cp in copies:
                cp.wait()

            def accum(sy, c2):
                syv = jnp.zeros((L,), jnp.int32) + sy
                ay0s = _dyn_gather(ay0, syv)
                ay1s = _dyn_gather(ay1, syv)
                bin_base = (sy >> 1) * PW
                row0 = sy * SX
                for sx in range(SX):
                    bx0s = _splat(bx0, sx)
                    bx1s = _splat(bx1, sx)
                    w00 = ay0s * bx0s
                    w01 = ay0s * bx1s
                    w10 = ay1s * bx0s
                    w11 = ay1s * bx1s
                    bin_ = bin_base + (sx >> 1)
                    s0 = jnp.zeros((L,), jnp.int32) + (row0 + sx)
                    s1 = s0 + NS
                    for g in range(4):
                        v00 = plsc.load_gather(rows, [s0, cols[g]])
                        v01 = plsc.load_gather(rows, [s1, cols[g]])
                        v10 = plsc.load_gather(rows, [s0, cols[g] + C])
                        v11 = plsc.load_gather(rows, [s1, cols[g] + C])
                        acc = w00 * v00 + w01 * v01 + w10 * v10 + w11 * v11
                        plsc.addupdate_scatter(outbuf, [cgs[g] + bin_], acc)
                return c2

            lax.fori_loop(0, SY, accum, 0, unroll=False)

            roi = wid * RPW + r
            pltpu.sync_copy(outbuf, out_hbm.at[pl.ds(roi * OUTSZ, OUTSZ)])
            return carry

        lax.fori_loop(0, RPW, per_roi, 0, unroll=False)

    return k(table, rois_flat)


def kernel(features, rois):
    N, C, H, W = features.shape
    K = rois.shape[0]
    KPAD = 1024 if K <= 1024 else ((K + 255) // 256) * 256
    table = _make_table(features)
    rois_pad = jnp.zeros((KPAD, 8), jnp.float32).at[:K, :5].set(rois)
    out_flat = _roi_align_sc(table, rois_pad.reshape(-1),
                             N=N, C=C, H=H, W=W, KPAD=KPAD)
    out = out_flat.reshape(KPAD, C, OUT_SIZE, OUT_SIZE)
    return out[:K]
